# baseline (speedup 1.0000x reference)
import functools

import jax
import jax.numpy as jnp
from jax import lax
from jax.experimental import pallas as pl
from jax.experimental.pallas import tpu as pltpu
from jax.experimental.pallas import tpu_sc as plsc

B = 4096
L = 50
D = 64
V_U0 = 100000
V_U1 = 1000
V_I0 = 1000000
V_I1 = 100000
NC = 2
NS = 16
NW = NC * NS
BW = B // NW
HTOT = B * L
HW = HTOT // NW
SUB = 128
KSUB = 5
CH = SUB * KSUB
NCH = HW // CH


@functools.partial(
    pl.kernel,
    out_type=(
        jax.ShapeDtypeStruct((B, 4, D), jnp.float32),
        jax.ShapeDtypeStruct((HTOT, 2, D), jnp.float32),
        jax.ShapeDtypeStruct((HTOT, 2, D), jnp.float32),
    ),
    mesh=plsc.VectorSubcoreMesh(core_axis_name="c", subcore_axis_name="s"),
    compiler_params=pltpu.CompilerParams(use_tc_tiling_on_sc=False),
    scratch_types=(
        pltpu.VMEM((BW,), jnp.int32),
        pltpu.VMEM((BW, D), jnp.float32),
        pltpu.VMEM((KSUB, SUB), jnp.int32),
        pltpu.VMEM((KSUB, SUB), jnp.int32),
        pltpu.VMEM((CH, D), jnp.float32),
        pltpu.VMEM((CH, D), jnp.float32),
        pltpu.SemaphoreType.DMA,
        pltpu.SemaphoreType.DMA,
        pltpu.SemaphoreType.DMA,
        pltpu.SemaphoreType.DMA,
        pltpu.SemaphoreType.DMA,
    ),
)
def _embed_sc(u0_ids, u1_ids, t0_ids, t1_ids,
              p0_ids, p1_ids, n0_ids, n1_ids,
              u0_tab, u1_tab, i0_tab, i1_tab,
              ut_out, pos_out, neg_out,
              uidx, urows, hidx0, hidx1, hrows0, hrows1,
              usem, gsem0, gsem1, wsem0, wsem1):
    wid = lax.axis_index("s") * NC + lax.axis_index("c")
    hidx = (hidx0, hidx1)
    hrows = (hrows0, hrows1)
    gsems = (gsem0, gsem1)
    wsems = (wsem0, wsem1)

    ubase = wid * BW
    for f, (ids, tab) in enumerate(((u0_ids, u0_tab), (u1_ids, u1_tab),
                                    (t0_ids, i0_tab), (t1_ids, i1_tab))):
        pltpu.sync_copy(ids.at[pl.ds(ubase, BW)], uidx)
        pltpu.async_copy(tab.at[uidx], urows, usem).wait()
        pltpu.sync_copy(urows, ut_out.at[pl.ds(ubase, BW), f])

    nrow_w = HW // SUB
    base_row = wid * nrow_w

    for ids2, tab, out, f in ((p0_ids, i0_tab, pos_out, 0),
                              (p1_ids, i1_tab, pos_out, 1),
                              (n0_ids, i0_tab, neg_out, 0),
                              (n1_ids, i1_tab, neg_out, 1)):
        def stage_fire(c, b, ids2=ids2, tab=tab):
            row0 = base_row + c * KSUB
            pltpu.sync_copy(ids2.at[pl.ds(row0, KSUB)], hidx[b])
            for j in range(KSUB):
                pltpu.async_copy(tab.at[hidx[b].at[j]],
                                 hrows[b].at[pl.ds(j * SUB, SUB)], gsems[b])

        def wait_gather(b, tab=tab):
            for j in range(KSUB):
                pltpu.make_async_copy(tab.at[hidx[b].at[j]],
                                      hrows[b].at[pl.ds(j * SUB, SUB)],
                                      gsems[b]).wait()

        def write_async(c, b, out=out, f=f):
            row0 = base_row + c * KSUB
            pltpu.async_copy(hrows[b], out.at[pl.ds(row0 * SUB, CH), f],
                             wsems[b])

        def wait_write(b, out=out, f=f):
            pltpu.make_async_copy(hrows[b],
                                  out.at[pl.ds(base_row * SUB, CH), f],
                                  wsems[b]).wait()

        for b in range(2):
            stage_fire(b, b)

        def body(i, carry):
            for b in range(2):
                c = 2 * i + b
                wait_gather(b)
                write_async(c, b)
                nxt = c + 2

                @pl.when(nxt < NCH)
                def _(b=b, nxt=nxt):
                    wait_write(b)
                    stage_fire(nxt, b)
            return carry

        lax.fori_loop(0, NCH // 2, body, 0)
        wait_write(0)
        wait_write(1)


def _tr_body(in_ref, out_ref, *, half):
    x = in_ref[...]
    eye = jnp.eye(64, dtype=jnp.float32)
    dn = (((0,), (0,)), ((), ()))
    a = jax.lax.dot_general(x[:, :half], eye, dn,
                            preferred_element_type=jnp.float32)
    b = jax.lax.dot_general(x[:, half:], eye, dn,
                            preferred_element_type=jnp.float32)
    out_ref[...] = jnp.concatenate([a, b], axis=1)


def _tr_body2(in_a, in_b, out_a, out_b, *, half):
    ta = in_a[...].T
    out_a[...] = jnp.concatenate([ta[:half], ta[half:]], axis=1)
    tb = in_b[...].T
    out_b[...] = jnp.concatenate([tb[:half], tb[half:]], axis=1)


def _transpose_tables2(tab_a, tab_b, vcard, bv):
    ta = jnp.swapaxes(tab_a, 0, 1)
    tb = jnp.swapaxes(tab_b, 0, 1)
    nb = -(-vcard // bv)
    spec_in = pl.BlockSpec((64, bv), lambda i: (0, i))
    spec_out = pl.BlockSpec((bv // 2, 128), lambda i: (i, 0))
    shp = jax.ShapeDtypeStruct((nb * bv // 2, 128), jnp.float32)
    oa, ob = pl.pallas_call(
        functools.partial(_tr_body2, half=bv // 2),
        grid=(nb,),
        in_specs=[spec_in, spec_in],
        out_specs=[spec_out, spec_out],
        out_shape=[shp, shp],
    )(ta, tb)
    return oa.reshape(nb * bv, 64), ob.reshape(nb * bv, 64)


def _transpose_table(tab, vcard, bv):
    tt = jnp.swapaxes(tab, 0, 1)
    nb = -(-vcard // bv)
    out = pl.pallas_call(
        functools.partial(_tr_body, half=bv // 2),
        grid=(nb,),
        in_specs=[pl.BlockSpec((64, bv), lambda i: (0, i))],
        out_specs=pl.BlockSpec((bv // 2, 128), lambda i: (i, 0)),
        out_shape=jax.ShapeDtypeStruct((nb * bv // 2, 128), jnp.float32),
    )(tt)
    return out.reshape(nb * bv, 64)


def _tr_idx(v, bv):
    p = v % bv
    return (v - p) + 2 * (p % (bv // 2)) + (p // (bv // 2))


def kernel(user_feat0_ids, user_feat1_ids, target_item0_ids, target_item1_ids,
           pos_hist_item0_ids, pos_hist_item1_ids,
           neg_hist_item0_ids, neg_hist_item1_ids,
           user0_table, user1_table, item0_table, item1_table):
    def _hist(ids, bv):
        return jnp.swapaxes(_tr_idx(ids, bv), 0, 1).reshape(HTOT // SUB, SUB)

    p0 = _hist(pos_hist_item0_ids, 8192)
    p1 = _hist(pos_hist_item1_ids, 4096)
    n0 = _hist(neg_hist_item0_ids, 8192)
    n1 = _hist(neg_hist_item1_ids, 4096)
    u0t, i1t = _transpose_tables2(user0_table, item1_table, V_U0, 4096)
    u1t = _transpose_table(user1_table, V_U1, 1024)
    i0t = _transpose_table(item0_table, V_I0, 8192)
    ut, pos, neg = _embed_sc(_tr_idx(user_feat0_ids, 4096),
                             _tr_idx(user_feat1_ids, 1024),
                             _tr_idx(target_item0_ids, 8192),
                             _tr_idx(target_item1_ids, 4096),
                             p0, p1, n0, n1,
                             u0t, u1t, i0t, i1t)
    pos = jnp.swapaxes(pos.reshape(L, B, 2 * D), 0, 1)
    neg = jnp.swapaxes(neg.reshape(L, B, 2 * D), 0, 1)
    return (ut.reshape(B, 4 * D), pos, neg)


# speedup vs baseline: 1.0698x; 1.0698x over previous
import functools

import jax
import jax.numpy as jnp
from jax import lax
from jax.experimental import pallas as pl
from jax.experimental.pallas import tpu as pltpu
from jax.experimental.pallas import tpu_sc as plsc

B = 4096
L = 50
D = 64
V_U0 = 100000
V_U1 = 1000
V_I0 = 1000000
V_I1 = 100000
NC = 2
NS = 16
NW = NC * NS
BW = B // NW
HTOT = B * L
HW = HTOT // NW
SUB = 128
KSUB = 5
CH = SUB * KSUB
NCH = HW // CH


@functools.partial(
    pl.kernel,
    out_type=(
        jax.ShapeDtypeStruct((B, 4, D), jnp.float32),
        jax.ShapeDtypeStruct((HTOT, 2, D), jnp.float32),
        jax.ShapeDtypeStruct((HTOT, 2, D), jnp.float32),
    ),
    mesh=plsc.VectorSubcoreMesh(core_axis_name="c", subcore_axis_name="s"),
    compiler_params=pltpu.CompilerParams(use_tc_tiling_on_sc=False),
    scratch_types=(
        pltpu.VMEM((BW,), jnp.int32),
        pltpu.VMEM((BW, D), jnp.float32),
        pltpu.VMEM((KSUB, SUB), jnp.int32),
        pltpu.VMEM((KSUB, SUB), jnp.int32),
        pltpu.VMEM((CH, D), jnp.float32),
        pltpu.VMEM((CH, D), jnp.float32),
        pltpu.SemaphoreType.DMA,
        pltpu.SemaphoreType.DMA,
        pltpu.SemaphoreType.DMA,
        pltpu.SemaphoreType.DMA,
        pltpu.SemaphoreType.DMA,
    ),
)
def _embed_sc(u0_ids, u1_ids, t0_ids, t1_ids,
              p0_ids, p1_ids, n0_ids, n1_ids,
              u0_tab, u1_tab, i0_tab, i1_tab,
              ut_out, pos_out, neg_out,
              uidx, urows, hidx0, hidx1, hrows0, hrows1,
              usem, gsem0, gsem1, wsem0, wsem1):
    wid = lax.axis_index("s") * NC + lax.axis_index("c")
    hidx = (hidx0, hidx1)
    hrows = (hrows0, hrows1)
    gsems = (gsem0, gsem1)
    wsems = (wsem0, wsem1)

    ubase = wid * BW
    for f, (ids, tab) in enumerate(((u0_ids, u0_tab), (u1_ids, u1_tab),
                                    (t0_ids, i0_tab), (t1_ids, i1_tab))):
        pltpu.sync_copy(ids.at[pl.ds(ubase, BW)], uidx)
        pltpu.async_copy(tab.at[uidx], urows, usem).wait()
        pltpu.sync_copy(urows, ut_out.at[pl.ds(ubase, BW), f])

    nrow_w = HW // SUB
    base_row = wid * nrow_w

    for ids2, tab, out, f in ((p0_ids, i0_tab, pos_out, 0),
                              (p1_ids, i1_tab, pos_out, 1),
                              (n0_ids, i0_tab, neg_out, 0),
                              (n1_ids, i1_tab, neg_out, 1)):
        def stage_fire(c, b, ids2=ids2, tab=tab):
            row0 = base_row + c * KSUB
            pltpu.sync_copy(ids2.at[pl.ds(row0, KSUB)], hidx[b])
            for j in range(KSUB):
                pltpu.async_copy(tab.at[hidx[b].at[j]],
                                 hrows[b].at[pl.ds(j * SUB, SUB)], gsems[b])

        def wait_gather(b, tab=tab):
            for j in range(KSUB):
                pltpu.make_async_copy(tab.at[hidx[b].at[j]],
                                      hrows[b].at[pl.ds(j * SUB, SUB)],
                                      gsems[b]).wait()

        def write_async(c, b, out=out, f=f):
            row0 = base_row + c * KSUB
            pltpu.async_copy(hrows[b], out.at[pl.ds(row0 * SUB, CH), f],
                             wsems[b])

        def wait_write(b, out=out, f=f):
            pltpu.make_async_copy(hrows[b],
                                  out.at[pl.ds(base_row * SUB, CH), f],
                                  wsems[b]).wait()

        for b in range(2):
            stage_fire(b, b)

        def body(i, carry):
            for b in range(2):
                c = 2 * i + b
                wait_gather(b)
                write_async(c, b)
                nxt = c + 2

                @pl.when(nxt < NCH)
                def _(b=b, nxt=nxt):
                    wait_write(b)
                    stage_fire(nxt, b)
            return carry

        lax.fori_loop(0, NCH // 2, body, 0)
        wait_write(0)
        wait_write(1)


def _tr_body(in_ref, out_ref, *, half):
    t = in_ref[...].T
    out_ref[...] = jnp.concatenate([t[:half], t[half:]], axis=1)


def _tr_body2(in_a, in_b, out_a, out_b, *, half):
    ta = in_a[...].T
    out_a[...] = jnp.concatenate([ta[:half], ta[half:]], axis=1)
    tb = in_b[...].T
    out_b[...] = jnp.concatenate([tb[:half], tb[half:]], axis=1)


def _transpose_tables2(tab_a, tab_b, vcard, bv):
    ta = jnp.swapaxes(tab_a, 0, 1)
    tb = jnp.swapaxes(tab_b, 0, 1)
    nb = -(-vcard // bv)
    spec_in = pl.BlockSpec((64, bv), lambda i: (0, i))
    spec_out = pl.BlockSpec((bv // 2, 128), lambda i: (i, 0))
    shp = jax.ShapeDtypeStruct((nb * bv // 2, 128), jnp.float32)
    oa, ob = pl.pallas_call(
        functools.partial(_tr_body2, half=bv // 2),
        grid=(nb,),
        in_specs=[spec_in, spec_in],
        out_specs=[spec_out, spec_out],
        out_shape=[shp, shp],
    )(ta, tb)
    return oa.reshape(nb * bv, 64), ob.reshape(nb * bv, 64)


def _transpose_table(tab, vcard, bv):
    tt = jnp.swapaxes(tab, 0, 1)
    nb = -(-vcard // bv)
    out = pl.pallas_call(
        functools.partial(_tr_body, half=bv // 2),
        grid=(nb,),
        in_specs=[pl.BlockSpec((64, bv), lambda i: (0, i))],
        out_specs=pl.BlockSpec((bv // 2, 128), lambda i: (i, 0)),
        out_shape=jax.ShapeDtypeStruct((nb * bv // 2, 128), jnp.float32),
    )(tt)
    return out.reshape(nb * bv, 64)


def _tr_idx(v, bv):
    p = v % bv
    return (v - p) + 2 * (p % (bv // 2)) + (p // (bv // 2))


def kernel(user_feat0_ids, user_feat1_ids, target_item0_ids, target_item1_ids,
           pos_hist_item0_ids, pos_hist_item1_ids,
           neg_hist_item0_ids, neg_hist_item1_ids,
           user0_table, user1_table, item0_table, item1_table):
    def _hist(ids, bv):
        return jnp.swapaxes(_tr_idx(ids, bv), 0, 1).reshape(HTOT // SUB, SUB)

    p0 = _hist(pos_hist_item0_ids, 16384)
    p1 = _hist(pos_hist_item1_ids, 4096)
    n0 = _hist(neg_hist_item0_ids, 16384)
    n1 = _hist(neg_hist_item1_ids, 4096)
    u0t, i1t = _transpose_tables2(user0_table, item1_table, V_U0, 4096)
    u1t = _transpose_table(user1_table, V_U1, 1024)
    i0t = _transpose_table(item0_table, V_I0, 16384)
    ut, pos, neg = _embed_sc(_tr_idx(user_feat0_ids, 4096),
                             _tr_idx(user_feat1_ids, 1024),
                             _tr_idx(target_item0_ids, 16384),
                             _tr_idx(target_item1_ids, 4096),
                             p0, p1, n0, n1,
                             u0t, u1t, i0t, i1t)
    pos = jnp.swapaxes(pos.reshape(L, B, 2 * D), 0, 1)
    neg = jnp.swapaxes(neg.reshape(L, B, 2 * D), 0, 1)
    return (ut.reshape(B, 4 * D), pos, neg)


# speedup vs baseline: 1.1007x; 1.0289x over previous
import functools

import jax
import jax.numpy as jnp
from jax import lax
from jax.experimental import pallas as pl
from jax.experimental.pallas import tpu as pltpu
from jax.experimental.pallas import tpu_sc as plsc

B = 4096
L = 50
D = 64
V_U0 = 100000
V_U1 = 1000
V_I0 = 1000000
V_I1 = 100000
NC = 2
NS = 16
NW = NC * NS
BW = B // NW
HTOT = B * L
HW = HTOT // NW
SUB = 128
KSUB = 5
CH = SUB * KSUB
NCH = HW // CH


@functools.partial(
    pl.kernel,
    out_type=(
        jax.ShapeDtypeStruct((B, 4, D), jnp.float32),
        jax.ShapeDtypeStruct((HTOT, 2, D), jnp.float32),
        jax.ShapeDtypeStruct((HTOT, 2, D), jnp.float32),
    ),
    mesh=plsc.VectorSubcoreMesh(core_axis_name="c", subcore_axis_name="s"),
    compiler_params=pltpu.CompilerParams(use_tc_tiling_on_sc=False),
    scratch_types=(
        pltpu.VMEM((BW,), jnp.int32),
        pltpu.VMEM((BW, D), jnp.float32),
        pltpu.VMEM((KSUB, SUB), jnp.int32),
        pltpu.VMEM((KSUB, SUB), jnp.int32),
        pltpu.VMEM((CH, D), jnp.float32),
        pltpu.VMEM((CH, D), jnp.float32),
        pltpu.SemaphoreType.DMA,
        pltpu.SemaphoreType.DMA,
        pltpu.SemaphoreType.DMA,
        pltpu.SemaphoreType.DMA,
        pltpu.SemaphoreType.DMA,
    ),
)
def _embed_sc(u0_ids, u1_ids, t0_ids, t1_ids,
              p0_ids, p1_ids, n0_ids, n1_ids,
              u0_tab, u1_tab, i0_tab, i1_tab,
              ut_out, pos_out, neg_out,
              uidx, urows, hidx0, hidx1, hrows0, hrows1,
              usem, gsem0, gsem1, wsem0, wsem1):
    wid = lax.axis_index("s") * NC + lax.axis_index("c")
    hidx = (hidx0, hidx1)
    hrows = (hrows0, hrows1)
    gsems = (gsem0, gsem1)
    wsems = (wsem0, wsem1)

    ubase = wid * BW
    for f, (ids, tab) in enumerate(((u0_ids, u0_tab), (u1_ids, u1_tab),
                                    (t0_ids, i0_tab), (t1_ids, i1_tab))):
        pltpu.sync_copy(ids.at[pl.ds(ubase, BW)], uidx)
        pltpu.async_copy(tab.at[uidx], urows, usem).wait()
        pltpu.sync_copy(urows, ut_out.at[pl.ds(ubase, BW), f])

    nrow_w = HW // SUB
    base_row = wid * nrow_w

    for ids2, tab, out, f in ((p0_ids, i0_tab, pos_out, 0),
                              (p1_ids, i1_tab, pos_out, 1),
                              (n0_ids, i0_tab, neg_out, 0),
                              (n1_ids, i1_tab, neg_out, 1)):
        def stage_fire(c, b, ids2=ids2, tab=tab):
            row0 = base_row + c * KSUB
            pltpu.sync_copy(ids2.at[pl.ds(row0, KSUB)], hidx[b])
            for j in range(KSUB):
                pltpu.async_copy(tab.at[hidx[b].at[j]],
                                 hrows[b].at[pl.ds(j * SUB, SUB)], gsems[b])

        def wait_gather(b, tab=tab):
            for j in range(KSUB):
                pltpu.make_async_copy(tab.at[hidx[b].at[j]],
                                      hrows[b].at[pl.ds(j * SUB, SUB)],
                                      gsems[b]).wait()

        def write_async(c, b, out=out, f=f):
            row0 = base_row + c * KSUB
            pltpu.async_copy(hrows[b], out.at[pl.ds(row0 * SUB, CH), f],
                             wsems[b])

        def wait_write(b, out=out, f=f):
            pltpu.make_async_copy(hrows[b],
                                  out.at[pl.ds(base_row * SUB, CH), f],
                                  wsems[b]).wait()

        for b in range(2):
            stage_fire(b, b)

        def body(i, carry):
            for b in range(2):
                c = 2 * i + b
                wait_gather(b)
                write_async(c, b)
                nxt = c + 2

                @pl.when(nxt < NCH)
                def _(b=b, nxt=nxt):
                    wait_write(b)
                    stage_fire(nxt, b)
            return carry

        lax.fori_loop(0, NCH // 2, body, 0)
        wait_write(0)
        wait_write(1)


def _tr_body(in_ref, out_ref, *, half):
    t = in_ref[...].T
    out_ref[...] = jnp.concatenate([t[:half], t[half:]], axis=1)


def _tr_body2(in_a, in_b, out_a, out_b, *, half):
    ta = in_a[...].T
    out_a[...] = jnp.concatenate([ta[:half], ta[half:]], axis=1)
    tb = in_b[...].T
    out_b[...] = jnp.concatenate([tb[:half], tb[half:]], axis=1)


def _transpose_tables2(tab_a, tab_b, vcard, bv):
    ta = jnp.swapaxes(tab_a, 0, 1)
    tb = jnp.swapaxes(tab_b, 0, 1)
    nb = -(-vcard // bv)
    spec_in = pl.BlockSpec((64, bv), lambda i: (0, i))
    spec_out = pl.BlockSpec((bv // 2, 128), lambda i: (i, 0))
    shp = jax.ShapeDtypeStruct((nb * bv // 2, 128), jnp.float32)
    oa, ob = pl.pallas_call(
        functools.partial(_tr_body2, half=bv // 2),
        grid=(nb,),
        in_specs=[spec_in, spec_in],
        out_specs=[spec_out, spec_out],
        out_shape=[shp, shp],
    )(ta, tb)
    return oa.reshape(nb * bv, 64), ob.reshape(nb * bv, 64)


def _transpose_table(tab, vcard, bv):
    tt = jnp.swapaxes(tab, 0, 1)
    nb = -(-vcard // bv)
    out = pl.pallas_call(
        functools.partial(_tr_body, half=bv // 2),
        grid=(nb,),
        in_specs=[pl.BlockSpec((64, bv), lambda i: (0, i))],
        out_specs=pl.BlockSpec((bv // 2, 128), lambda i: (i, 0)),
        out_shape=jax.ShapeDtypeStruct((nb * bv // 2, 128), jnp.float32),
    )(tt)
    return out.reshape(nb * bv, 64)


def _tr_idx(v, bv):
    p = v % bv
    return (v - p) + 2 * (p % (bv // 2)) + (p // (bv // 2))


def kernel(user_feat0_ids, user_feat1_ids, target_item0_ids, target_item1_ids,
           pos_hist_item0_ids, pos_hist_item1_ids,
           neg_hist_item0_ids, neg_hist_item1_ids,
           user0_table, user1_table, item0_table, item1_table):
    def _hist(ids, bv):
        return jnp.swapaxes(_tr_idx(ids, bv), 0, 1).reshape(HTOT // SUB, SUB)

    p0 = _hist(pos_hist_item0_ids, 32768)
    p1 = _hist(pos_hist_item1_ids, 4096)
    n0 = _hist(neg_hist_item0_ids, 32768)
    n1 = _hist(neg_hist_item1_ids, 4096)
    u0t, i1t = _transpose_tables2(user0_table, item1_table, V_U0, 4096)
    u1t = _transpose_table(user1_table, V_U1, 1024)
    i0t = _transpose_table(item0_table, V_I0, 32768)
    ut, pos, neg = _embed_sc(_tr_idx(user_feat0_ids, 4096),
                             _tr_idx(user_feat1_ids, 1024),
                             _tr_idx(target_item0_ids, 32768),
                             _tr_idx(target_item1_ids, 4096),
                             p0, p1, n0, n1,
                             u0t, u1t, i0t, i1t)
    pos = jnp.swapaxes(pos.reshape(L, B, 2 * D), 0, 1)
    neg = jnp.swapaxes(neg.reshape(L, B, 2 * D), 0, 1)
    return (ut.reshape(B, 4 * D), pos, neg)


# speedup vs baseline: 1.1057x; 1.0045x over previous
import functools

import jax
import jax.numpy as jnp
from jax import lax
from jax.experimental import pallas as pl
from jax.experimental.pallas import tpu as pltpu
from jax.experimental.pallas import tpu_sc as plsc

B = 4096
L = 50
D = 64
V_U0 = 100000
V_U1 = 1000
V_I0 = 1000000
V_I1 = 100000
NC = 2
NS = 16
NW = NC * NS
BW = B // NW
HTOT = B * L
HW = HTOT // NW
SUB = 128
KSUB = 5
CH = SUB * KSUB
NCH = HW // CH


@functools.partial(
    pl.kernel,
    out_type=(
        jax.ShapeDtypeStruct((B, 4, D), jnp.float32),
        jax.ShapeDtypeStruct((HTOT, 2, D), jnp.float32),
        jax.ShapeDtypeStruct((HTOT, 2, D), jnp.float32),
    ),
    mesh=plsc.VectorSubcoreMesh(core_axis_name="c", subcore_axis_name="s"),
    compiler_params=pltpu.CompilerParams(use_tc_tiling_on_sc=False),
    scratch_types=(
        pltpu.VMEM((BW,), jnp.int32),
        pltpu.VMEM((BW, D), jnp.float32),
        pltpu.VMEM((KSUB, SUB), jnp.int32),
        pltpu.VMEM((KSUB, SUB), jnp.int32),
        pltpu.VMEM((CH, D), jnp.float32),
        pltpu.VMEM((CH, D), jnp.float32),
        pltpu.SemaphoreType.DMA,
        pltpu.SemaphoreType.DMA,
        pltpu.SemaphoreType.DMA,
        pltpu.SemaphoreType.DMA,
        pltpu.SemaphoreType.DMA,
    ),
)
def _embed_sc(u0_ids, u1_ids, t0_ids, t1_ids,
              p0_ids, p1_ids, n0_ids, n1_ids,
              u0_tab, u1_tab, i0_tab, i1_tab,
              ut_out, pos_out, neg_out,
              uidx, urows, hidx0, hidx1, hrows0, hrows1,
              usem, gsem0, gsem1, wsem0, wsem1):
    wid = lax.axis_index("s") * NC + lax.axis_index("c")
    hidx = (hidx0, hidx1)
    hrows = (hrows0, hrows1)
    gsems = (gsem0, gsem1)
    wsems = (wsem0, wsem1)

    ubase = wid * BW
    for f, (ids, tab) in enumerate(((u0_ids, u0_tab), (u1_ids, u1_tab),
                                    (t0_ids, i0_tab), (t1_ids, i1_tab))):
        pltpu.sync_copy(ids.at[pl.ds(ubase, BW)], uidx)
        pltpu.async_copy(tab.at[uidx], urows, usem).wait()
        pltpu.sync_copy(urows, ut_out.at[pl.ds(ubase, BW), f])

    nrow_w = HW // SUB
    base_row = wid * nrow_w

    for ids2, tab, out, f in ((p0_ids, i0_tab, pos_out, 0),
                              (p1_ids, i1_tab, pos_out, 1),
                              (n0_ids, i0_tab, neg_out, 0),
                              (n1_ids, i1_tab, neg_out, 1)):
        def stage_fire(c, b, ids2=ids2, tab=tab):
            row0 = base_row + c * KSUB
            pltpu.sync_copy(ids2.at[pl.ds(row0, KSUB)], hidx[b])
            for j in range(KSUB):
                pltpu.async_copy(tab.at[hidx[b].at[j]],
                                 hrows[b].at[pl.ds(j * SUB, SUB)], gsems[b])

        def wait_gather(b, tab=tab):
            for j in range(KSUB):
                pltpu.make_async_copy(tab.at[hidx[b].at[j]],
                                      hrows[b].at[pl.ds(j * SUB, SUB)],
                                      gsems[b]).wait()

        def write_async(c, b, out=out, f=f):
            row0 = base_row + c * KSUB
            pltpu.async_copy(hrows[b], out.at[pl.ds(row0 * SUB, CH), f],
                             wsems[b])

        def wait_write(b, out=out, f=f):
            pltpu.make_async_copy(hrows[b],
                                  out.at[pl.ds(base_row * SUB, CH), f],
                                  wsems[b]).wait()

        for b in range(2):
            stage_fire(b, b)

        def body(i, carry):
            for b in range(2):
                c = 2 * i + b
                wait_gather(b)
                write_async(c, b)
                nxt = c + 2

                @pl.when(nxt < NCH)
                def _(b=b, nxt=nxt):
                    wait_write(b)
                    stage_fire(nxt, b)
            return carry

        lax.fori_loop(0, NCH // 2, body, 0)
        wait_write(0)
        wait_write(1)


def _tr_body(in_ref, out_ref, *, half):
    t = in_ref[...].T
    out_ref[...] = jnp.concatenate([t[:half], t[half:]], axis=1)


def _tr_body2(in_a, in_b, out_a, out_b, *, half):
    ta = in_a[...].T
    out_a[...] = jnp.concatenate([ta[:half], ta[half:]], axis=1)
    tb = in_b[...].T
    out_b[...] = jnp.concatenate([tb[:half], tb[half:]], axis=1)


def _transpose_tables2(tab_a, tab_b, vcard, bv):
    ta = jnp.swapaxes(tab_a, 0, 1)
    tb = jnp.swapaxes(tab_b, 0, 1)
    nb = -(-vcard // bv)
    spec_in = pl.BlockSpec((64, bv), lambda i: (0, i))
    spec_out = pl.BlockSpec((bv // 2, 128), lambda i: (i, 0))
    shp = jax.ShapeDtypeStruct((nb * bv // 2, 128), jnp.float32)
    oa, ob = pl.pallas_call(
        functools.partial(_tr_body2, half=bv // 2),
        grid=(nb,),
        in_specs=[spec_in, spec_in],
        out_specs=[spec_out, spec_out],
        out_shape=[shp, shp],
    )(ta, tb)
    return oa.reshape(nb * bv, 64), ob.reshape(nb * bv, 64)


def _transpose_table(tab, vcard, bv):
    tt = jnp.swapaxes(tab, 0, 1)
    nb = -(-vcard // bv)
    out = pl.pallas_call(
        functools.partial(_tr_body, half=bv // 2),
        grid=(nb,),
        in_specs=[pl.BlockSpec((64, bv), lambda i: (0, i))],
        out_specs=pl.BlockSpec((bv // 2, 128), lambda i: (i, 0)),
        out_shape=jax.ShapeDtypeStruct((nb * bv // 2, 128), jnp.float32),
    )(tt)
    return out.reshape(nb * bv, 64)


def _tr_idx(v, bv):
    p = v % bv
    return (v - p) + 2 * (p % (bv // 2)) + (p // (bv // 2))


def kernel(user_feat0_ids, user_feat1_ids, target_item0_ids, target_item1_ids,
           pos_hist_item0_ids, pos_hist_item1_ids,
           neg_hist_item0_ids, neg_hist_item1_ids,
           user0_table, user1_table, item0_table, item1_table):
    def _hist(ids, bv):
        return jnp.swapaxes(_tr_idx(ids, bv), 0, 1).reshape(HTOT // SUB, SUB)

    p0 = _hist(pos_hist_item0_ids, 32768)
    p1 = _hist(pos_hist_item1_ids, 16384)
    n0 = _hist(neg_hist_item0_ids, 32768)
    n1 = _hist(neg_hist_item1_ids, 16384)
    u0t, i1t = _transpose_tables2(user0_table, item1_table, V_U0, 16384)
    u1t = _transpose_table(user1_table, V_U1, 1024)
    i0t = _transpose_table(item0_table, V_I0, 32768)
    ut, pos, neg = _embed_sc(_tr_idx(user_feat0_ids, 16384),
                             _tr_idx(user_feat1_ids, 1024),
                             _tr_idx(target_item0_ids, 32768),
                             _tr_idx(target_item1_ids, 16384),
                             p0, p1, n0, n1,
                             u0t, u1t, i0t, i1t)
    pos = jnp.swapaxes(pos.reshape(L, B, 2 * D), 0, 1)
    neg = jnp.swapaxes(neg.reshape(L, B, 2 * D), 0, 1)
    return (ut.reshape(B, 4 * D), pos, neg)


# speedup vs baseline: 1.1852x; 1.0720x over previous
import functools

import jax
import jax.numpy as jnp
from jax import lax
from jax.experimental import pallas as pl
from jax.experimental.pallas import tpu as pltpu
from jax.experimental.pallas import tpu_sc as plsc

B = 4096
L = 50
D = 64
V_U0 = 100000
V_U1 = 1000
V_I0 = 1000000
V_I1 = 100000
BV_BIG = 32768
BV_MED = 8192
BV_SML = 1024
NC = 2
NS = 16
NW = NC * NS
BW = B // NW
HTOT = B * L
HW = HTOT // NW
SUB = 128
KSUB = 5
CH = SUB * KSUB
NCH = HW // CH

_MESH = plsc.VectorSubcoreMesh(core_axis_name="c", subcore_axis_name="s")
_PARAMS = pltpu.CompilerParams(use_tc_tiling_on_sc=False)
_HIST_SCRATCH = (
    pltpu.VMEM((KSUB, SUB), jnp.int32),
    pltpu.VMEM((KSUB, SUB), jnp.int32),
    pltpu.VMEM((CH, D), jnp.float32),
    pltpu.VMEM((CH, D), jnp.float32),
    pltpu.SemaphoreType.DMA,
    pltpu.SemaphoreType.DMA,
    pltpu.SemaphoreType.DMA,
    pltpu.SemaphoreType.DMA,
)


def _hist_set(ids2, tab, out, f, wid, hidx, hrows, gsems, wsems):
    base_row = wid * (HW // SUB)

    def stage_fire(c, b):
        row0 = base_row + c * KSUB
        pltpu.sync_copy(ids2.at[pl.ds(row0, KSUB)], hidx[b])
        for j in range(KSUB):
            pltpu.async_copy(tab.at[hidx[b].at[j]],
                             hrows[b].at[pl.ds(j * SUB, SUB)], gsems[b])

    def wait_gather(b):
        for j in range(KSUB):
            pltpu.make_async_copy(tab.at[hidx[b].at[j]],
                                  hrows[b].at[pl.ds(j * SUB, SUB)],
                                  gsems[b]).wait()

    def write_async(c, b):
        row0 = base_row + c * KSUB
        pltpu.async_copy(hrows[b], out.at[pl.ds(row0 * SUB, CH), f], wsems[b])

    def wait_write(b):
        pltpu.make_async_copy(hrows[b],
                              out.at[pl.ds(base_row * SUB, CH), f],
                              wsems[b]).wait()

    for b in range(2):
        stage_fire(b, b)

    def body(i, carry):
        for b in range(2):
            c = 2 * i + b
            wait_gather(b)
            write_async(c, b)
            nxt = c + 2

            @pl.when(nxt < NCH)
            def _(b=b, nxt=nxt):
                wait_write(b)
                stage_fire(nxt, b)
        return carry

    lax.fori_loop(0, NCH // 2, body, 0)
    wait_write(0)
    wait_write(1)


@functools.partial(
    pl.kernel,
    out_type=(),
    mesh=_MESH,
    compiler_params=_PARAMS,
    scratch_types=_HIST_SCRATCH,
)
def _sc_item1(p1_ids, n1_ids, i1_tab, pos_out, neg_out,
              hidx0, hidx1, hrows0, hrows1, gsem0, gsem1, wsem0, wsem1):
    wid = lax.axis_index("s") * NC + lax.axis_index("c")
    hidx = (hidx0, hidx1)
    hrows = (hrows0, hrows1)
    gsems = (gsem0, gsem1)
    wsems = (wsem0, wsem1)
    _hist_set(p1_ids, i1_tab, pos_out, 1, wid, hidx, hrows, gsems, wsems)
    _hist_set(n1_ids, i1_tab, neg_out, 1, wid, hidx, hrows, gsems, wsems)


@functools.partial(
    pl.kernel,
    out_type=jax.ShapeDtypeStruct((B, 4, D), jnp.float32),
    mesh=_MESH,
    compiler_params=_PARAMS,
    scratch_types=(
        pltpu.VMEM((BW,), jnp.int32),
        pltpu.VMEM((BW, D), jnp.float32),
    ) + _HIST_SCRATCH + (pltpu.SemaphoreType.DMA,),
)
def _sc_item0(u0_ids, u1_ids, t0_ids, t1_ids, p0_ids, n0_ids,
              u0_tab, u1_tab, i0_tab, i1_tab, pos_out, neg_out,
              ut_out,
              uidx, urows, hidx0, hidx1, hrows0, hrows1,
              gsem0, gsem1, wsem0, wsem1, usem):
    wid = lax.axis_index("s") * NC + lax.axis_index("c")
    hidx = (hidx0, hidx1)
    hrows = (hrows0, hrows1)
    gsems = (gsem0, gsem1)
    wsems = (wsem0, wsem1)

    ubase = wid * BW
    for f, (ids, tab) in enumerate(((u0_ids, u0_tab), (u1_ids, u1_tab),
                                    (t0_ids, i0_tab), (t1_ids, i1_tab))):
        pltpu.sync_copy(ids.at[pl.ds(ubase, BW)], uidx)
        pltpu.async_copy(tab.at[uidx], urows, usem).wait()
        pltpu.sync_copy(urows, ut_out.at[pl.ds(ubase, BW), f])

    _hist_set(p0_ids, i0_tab, pos_out, 0, wid, hidx, hrows, gsems, wsems)
    _hist_set(n0_ids, i0_tab, neg_out, 0, wid, hidx, hrows, gsems, wsems)


def _tr_body(in_ref, out_ref, *, half):
    t = in_ref[...].T
    out_ref[...] = jnp.concatenate([t[:half], t[half:]], axis=1)


def _tr_body2(in_a, in_b, out_a, out_b, *, half):
    ta = in_a[...].T
    out_a[...] = jnp.concatenate([ta[:half], ta[half:]], axis=1)
    tb = in_b[...].T
    out_b[...] = jnp.concatenate([tb[:half], tb[half:]], axis=1)


def _transpose_tables2(tab_a, tab_b, vcard, bv):
    ta = jnp.swapaxes(tab_a, 0, 1)
    tb = jnp.swapaxes(tab_b, 0, 1)
    nb = -(-vcard // bv)
    spec_in = pl.BlockSpec((64, bv), lambda i: (0, i))
    spec_out = pl.BlockSpec((bv // 2, 128), lambda i: (i, 0))
    shp = jax.ShapeDtypeStruct((nb * bv // 2, 128), jnp.float32)
    oa, ob = pl.pallas_call(
        functools.partial(_tr_body2, half=bv // 2),
        grid=(nb,),
        in_specs=[spec_in, spec_in],
        out_specs=[spec_out, spec_out],
        out_shape=[shp, shp],
    )(ta, tb)
    return oa.reshape(nb * bv, 64), ob.reshape(nb * bv, 64)


def _transpose_table(tab, vcard, bv):
    tt = jnp.swapaxes(tab, 0, 1)
    nb = -(-vcard // bv)
    out = pl.pallas_call(
        functools.partial(_tr_body, half=bv // 2),
        grid=(nb,),
        in_specs=[pl.BlockSpec((64, bv), lambda i: (0, i))],
        out_specs=pl.BlockSpec((bv // 2, 128), lambda i: (i, 0)),
        out_shape=jax.ShapeDtypeStruct((nb * bv // 2, 128), jnp.float32),
    )(tt)
    return out.reshape(nb * bv, 64)


def _tr_idx(v, bv):
    p = v % bv
    return (v - p) + 2 * (p % (bv // 2)) + (p // (bv // 2))


def kernel(user_feat0_ids, user_feat1_ids, target_item0_ids, target_item1_ids,
           pos_hist_item0_ids, pos_hist_item1_ids,
           neg_hist_item0_ids, neg_hist_item1_ids,
           user0_table, user1_table, item0_table, item1_table):
    def _hist(ids, bv):
        return jnp.swapaxes(_tr_idx(ids, bv), 0, 1).reshape(HTOT // SUB, SUB)

    p0 = _hist(pos_hist_item0_ids, BV_BIG)
    p1 = _hist(pos_hist_item1_ids, BV_MED)
    n0 = _hist(neg_hist_item0_ids, BV_BIG)
    n1 = _hist(neg_hist_item1_ids, BV_MED)
    u0t, i1t = _transpose_tables2(user0_table, item1_table, V_U0, BV_MED)
    u1t = _transpose_table(user1_table, V_U1, BV_SML)

    pos_ref = jax.new_ref(jax.lax.empty((HTOT, 2, D), jnp.float32))
    neg_ref = jax.new_ref(jax.lax.empty((HTOT, 2, D), jnp.float32))
    _sc_item1(p1, n1, i1t, pos_ref, neg_ref)

    i0t = _transpose_table(item0_table, V_I0, BV_BIG)
    ut = _sc_item0(_tr_idx(user_feat0_ids, BV_MED),
                   _tr_idx(user_feat1_ids, BV_SML),
                   _tr_idx(target_item0_ids, BV_BIG),
                   _tr_idx(target_item1_ids, BV_MED),
                   p0, n0, u0t, u1t, i0t, i1t, pos_ref, neg_ref)
    pos = jnp.swapaxes(pos_ref[...].reshape(L, B, 2 * D), 0, 1)
    neg = jnp.swapaxes(neg_ref[...].reshape(L, B, 2 * D), 0, 1)
    return (ut.reshape(B, 4 * D), pos, neg)
